# Initial kernel scaffold; baseline (speedup 1.0000x reference)
import jax
import jax.numpy as jnp
from jax.experimental import pallas as pl


def kernel(x, pos, edge_index, W):
    raise NotImplementedError("write your pallas kernel here")


# speedup vs baseline: 93.3585x; 93.3585x over previous
import functools

import jax
import jax.numpy as jnp
from jax import lax
from jax.experimental import pallas as pl
from jax.experimental.pallas import tpu as pltpu
from jax.experimental.pallas import tpu_sc as plsc

N = 50000
E = 1600000
OUT_CH = 32
MAX_VALUE = 2.0 * float(int(0.01 * 640 + 2) / 640)
SCALE = 1.0 / (2.0 * MAX_VALUE)

L = 16
NC = 2
NS = 16
NW = NC * NS

N_PAD = 51200
ZR = N_PAD // NS
E_PAD = NW * 51200
EPW = E_PAD // NW
K = 2048
NCHUNK = EPW // K
GROUPS = K // L

BN = 1024


_sc_mesh = plsc.VectorSubcoreMesh(core_axis_name="c", subcore_axis_name="s")


@functools.partial(
    pl.kernel,
    out_type=jax.ShapeDtypeStruct((NC * 2 * N_PAD,), jnp.float32),
    mesh=_sc_mesh,
    scratch_types=[
        pltpu.VMEM((K,), jnp.int32),
        pltpu.VMEM((K,), jnp.int32),
        pltpu.VMEM((K,), jnp.float32),
        pltpu.VMEM((K,), jnp.float32),
        pltpu.VMEM((K,), jnp.float32),
        pltpu.VMEM((K,), jnp.float32),
        pltpu.VMEM((K,), jnp.float32),
        pltpu.VMEM((K,), jnp.float32),
        pltpu.VMEM((K,), jnp.float32),
        pltpu.VMEM_SHARED((N_PAD,), jnp.float32),
        pltpu.VMEM_SHARED((N_PAD,), jnp.float32),
        pltpu.SemaphoreType.DMA,
        pltpu.SemaphoreType.DMA,
        pltpu.SemaphoreType.DMA,
        pltpu.SemaphoreType.DMA,
        pltpu.SemaphoreType.DMA,
    ],
)
def _sc_accumulate(px_h, py_h, xx_h, src_h, dst_h, zero_h, out_h,
                   sidx, didx, sx_v, sy_v, xx_v, dx_v, dy_v, a_v, b_v,
                   acc_a, acc_b, sem1, sem2, sem3, sem4, sem5):
    c = lax.axis_index("c")
    s = lax.axis_index("s")
    wid = c * NS + s

    pltpu.sync_copy(zero_h, acc_a.at[pl.ds(s * ZR, ZR)])
    pltpu.sync_copy(zero_h, acc_b.at[pl.ds(s * ZR, ZR)])
    plsc.subcore_barrier()

    def chunk_body(i, carry):
        base = wid * EPW + i * K
        pltpu.sync_copy(src_h.at[pl.ds(base, K)], sidx)
        pltpu.sync_copy(dst_h.at[pl.ds(base, K)], didx)
        cp1 = pltpu.async_copy(px_h.at[sidx], sx_v, sem1)
        cp2 = pltpu.async_copy(py_h.at[sidx], sy_v, sem2)
        cp3 = pltpu.async_copy(xx_h.at[sidx], xx_v, sem3)
        cp4 = pltpu.async_copy(px_h.at[didx], dx_v, sem4)
        cp5 = pltpu.async_copy(py_h.at[didx], dy_v, sem5)
        cp1.wait()
        cp2.wait()
        cp3.wait()
        cp4.wait()
        cp5.wait()

        def grp(j, carry2):
            sl = pl.ds(j * L, L)
            p = jnp.clip((dx_v[sl] - sx_v[sl]) * SCALE + 0.5, 0.0, 1.0)
            q = jnp.clip((dy_v[sl] - sy_v[sl]) * SCALE + 0.5, 0.0, 1.0)
            xv = xx_v[sl]
            a_v[sl] = xv * p
            b_v[sl] = xv * q
            return carry2

        lax.fori_loop(0, GROUPS, grp, None)
        pltpu.sync_copy(a_v, acc_a.at[didx], add=True)
        pltpu.sync_copy(b_v, acc_b.at[didx], add=True)
        return carry

    lax.fori_loop(0, NCHUNK, chunk_body, None)

    plsc.subcore_barrier()
    base_out = c * 2 * N_PAD + s * ZR
    pltpu.sync_copy(acc_a.at[pl.ds(s * ZR, ZR)],
                    out_h.at[pl.ds(base_out, ZR)])
    pltpu.sync_copy(acc_b.at[pl.ds(s * ZR, ZR)],
                    out_h.at[pl.ds(base_out + N_PAD, ZR)])


def _tc_expand_body(ab_ref, w4_ref, o_ref):
    o_ref[...] = lax.dot_general(
        ab_ref[...], w4_ref[...],
        dimension_numbers=(((0,), (0,)), ((), ())),
        preferred_element_type=jnp.float32)


_tc_expand = pl.pallas_call(
    _tc_expand_body,
    out_shape=jax.ShapeDtypeStruct((N_PAD, OUT_CH), jnp.float32),
    grid=(N_PAD // BN,),
    in_specs=[
        pl.BlockSpec((4, BN), lambda i: (0, i)),
        pl.BlockSpec((4, OUT_CH), lambda i: (0, 0)),
    ],
    out_specs=pl.BlockSpec((BN, OUT_CH), lambda i: (i, 0)),
)


@jax.jit
def kernel(x, pos, edge_index, W):
    px = jnp.pad(pos[:, 0], (0, N_PAD - N))
    py = jnp.pad(pos[:, 1], (0, N_PAD - N))
    xx = jnp.pad(x[:, 0], (0, N_PAD - N))
    pad_idx = jnp.full((E_PAD - E,), N, jnp.int32)
    src = jnp.concatenate([edge_index[0], pad_idx])
    dst = jnp.concatenate([edge_index[1], pad_idx])
    zeros = jnp.zeros((ZR,), jnp.float32)

    ab = _sc_accumulate(px, py, xx, src, dst, zeros)
    ab4 = ab.reshape(4, N_PAD)
    w4 = jnp.concatenate([W, W], axis=0)
    out = _tc_expand(ab4, w4)
    return out[:N]
